# baseline (speedup 1.0000x reference)
import jax
import jax.numpy as jnp
from jax.experimental import pallas as pl
from jax.experimental.pallas import tpu as pltpu

D0, D1, D2 = 1024, 1024, 128


def _gather_body(idx_ref, in_ref, out_ref, sem):
    idx = idx_ref[()]
    copy = pltpu.make_async_copy(in_ref.at[:, idx], out_ref, sem)
    copy.start()
    copy.wait()


def kernel(input, indices):
    idx = indices.astype(jnp.int32)
    return pl.pallas_call(
        _gather_body,
        in_specs=[
            pl.BlockSpec(memory_space=pltpu.SMEM),
            pl.BlockSpec(memory_space=pl.ANY),
        ],
        out_specs=pl.BlockSpec(memory_space=pltpu.VMEM),
        out_shape=jax.ShapeDtypeStruct((D0, D2), jnp.float32),
        scratch_shapes=[pltpu.SemaphoreType.DMA],
    )(idx, input)
